# baseline (speedup 1.0000x reference)
import functools

import jax
import jax.numpy as jnp
from jax import lax
from jax.experimental import pallas as pl
from jax.experimental.pallas import tpu as pltpu
from jax.experimental.pallas import tpu_sc as plsc

N_L = 4096
N_M = 16384
F1 = 64
F2 = 64
H = 128
_EPS_SQ = 1e-12
_BN_EPS = 1e-5

_QB = 512
_NQ = N_M // _QB
_MB = 512
_NB = N_M // _MB

_NW = 32
_QPW = N_M // _NW
_QCH = 256
_NCH = _QPW // _QCH
_RCH = 3 * _QCH


def _knn_body(q_ref, pt_ref, idx_ref, w_ref):
    q = q_ref[...]
    pt = pt_ref[...]
    d0 = q[:, 0:1] - pt[0:1, :]
    d1 = q[:, 1:2] - pt[1:2, :]
    d2 = q[:, 2:3] - pt[2:3, :]
    s = d0 * d0 + d1 * d1 + d2 * d2
    colsf = lax.broadcasted_iota(jnp.int32, s.shape, 1).astype(jnp.float32)
    big = jnp.float32(jnp.inf)
    m1 = jnp.min(s, axis=1, keepdims=True)
    m2 = jnp.min(jnp.where(s > m1, s, big), axis=1, keepdims=True)
    m3 = jnp.min(jnp.where(s > m2, s, big), axis=1, keepdims=True)
    ms = [m1, m2, m3]
    idxf = [jnp.min(jnp.where(s == m, colsf, big), axis=1, keepdims=True)
            for m in ms]
    w = [1.0 / jnp.maximum(m, jnp.float32(_EPS_SQ)) for m in ms]
    inv = 1.0 / (w[0] + w[1] + w[2])
    w_ref[...] = jnp.concatenate([wk * inv for wk in w], axis=1)
    idx_ref[...] = jnp.concatenate(
        [f.astype(jnp.int32) for f in idxf], axis=1)


def _knn(orig_coords, prop_t):
    return pl.pallas_call(
        _knn_body,
        grid=(_NQ,),
        in_specs=[pl.BlockSpec((_QB, 3), lambda i: (i, 0)),
                  pl.BlockSpec((3, N_L), lambda i: (0, 0))],
        out_specs=[pl.BlockSpec((_QB, 3), lambda i: (i, 0)),
                   pl.BlockSpec((_QB, 3), lambda i: (i, 0))],
        out_shape=[jax.ShapeDtypeStruct((N_M, 3), jnp.int32),
                   jax.ShapeDtypeStruct((N_M, 3), jnp.float32)],
    )(orig_coords, prop_t)


def _sc_interp(prop_feats, idx_flat, w_flat):
    mesh = plsc.VectorSubcoreMesh(core_axis_name="c", subcore_axis_name="s")

    @functools.partial(
        pl.kernel,
        mesh=mesh,
        out_type=jax.ShapeDtypeStruct((N_M, F1), jnp.float32),
        scratch_types=[
            pltpu.VMEM((_RCH,), jnp.int32),
            pltpu.VMEM((_RCH + 16,), jnp.float32),
            pltpu.VMEM((_RCH, F1), jnp.float32),
            pltpu.VMEM((_QCH, F1), jnp.float32),
            pltpu.SemaphoreType.DMA,
        ],
        compiler_params=pltpu.CompilerParams(use_tc_tiling_on_sc=False),
    )
    def k(feats_hbm, idx_hbm, wgt_hbm, out_hbm, idx_v, w_v, rows_v, out_v, sem):
        wid = lax.axis_index("s") * 2 + lax.axis_index("c")
        for ch in range(_NCH):
            qbase = wid * _QPW + ch * _QCH
            rbase = 3 * qbase
            pltpu.sync_copy(idx_hbm.at[pl.ds(rbase, _RCH)], idx_v)
            pltpu.sync_copy(wgt_hbm.at[pl.ds(rbase, _RCH)],
                            w_v.at[pl.ds(0, _RCH)])
            pltpu.async_copy(feats_hbm.at[idx_v], rows_v, sem).wait()

            def body(qq, carry):
                r = 3 * qq
                wv = w_v[pl.ds(r, 16)]
                w0 = wv[0]
                w1 = wv[1]
                w2 = wv[2]
                for cch in range(F1 // 16):
                    slc = pl.ds(cch * 16, 16)
                    out_v[qq, slc] = (rows_v[r, slc] * w0
                                      + rows_v[r + 1, slc] * w1
                                      + rows_v[r + 2, slc] * w2)
                return carry

            lax.fori_loop(0, _QCH, body, 0)
            pltpu.sync_copy(out_v, out_hbm.at[pl.ds(qbase, _QCH)])

    return k(prop_feats, idx_flat, w_flat)


def _bn(h, st, g, be):
    mu = st[0:1, :] * (1.0 / N_M)
    var = st[1:2, :] * (1.0 / N_M) - mu * mu
    v = var + jnp.float32(_BN_EPS)
    inv = lax.rsqrt(v)
    inv = inv * (1.5 - 0.5 * v * inv * inv)
    inv = inv * (1.5 - 0.5 * v * inv * inv)
    return jnp.maximum((h - mu) * inv * g + be, 0.0)


def _stats(h):
    return jnp.concatenate([jnp.sum(h, axis=0, keepdims=True),
                            jnp.sum(h * h, axis=0, keepdims=True)], axis=0)


def _mlp_body(c_ref, f_ref, it_ref, w0_ref, b0_ref, g0_ref, be0_ref,
              w1_ref, b1_ref, g1_ref, be1_ref, o_ref,
              h0_s, h1_s, acc0, acc1):
    step = pl.program_id(0)
    i = step % _NB
    rows = pl.ds(i * _MB, _MB)

    @pl.when(step < _NB)
    def _():
        x = jnp.concatenate([c_ref[...], f_ref[...], it_ref[...]], axis=1)
        h = jnp.dot(x, w0_ref[...],
                    preferred_element_type=jnp.float32) + b0_ref[...]
        h0_s[rows, :] = h

        @pl.when(step == 0)
        def _():
            acc0[...] = jnp.zeros_like(acc0)

        acc0[...] += _stats(h)

    @pl.when(jnp.logical_and(step >= _NB, step < 2 * _NB))
    def _():
        z = _bn(h0_s[rows, :], acc0[...], g0_ref[...], be0_ref[...])
        h1 = jnp.dot(z, w1_ref[...],
                     preferred_element_type=jnp.float32) + b1_ref[...]
        h1_s[rows, :] = h1

        @pl.when(step == _NB)
        def _():
            acc1[...] = jnp.zeros_like(acc1)

        acc1[...] += _stats(h1)

    @pl.when(step >= 2 * _NB)
    def _():
        o_ref[...] = _bn(h1_s[rows, :], acc1[...], g1_ref[...], be1_ref[...])


def _mlp(coords, feats, interp, w0, b0, g0, be0, w1, b1, g1, be1):
    blk = lambda g: (g % _NB, 0)
    cst = lambda g: (0, 0)
    out_map = lambda g: (jnp.where(g < 2 * _NB, 0, g % _NB), 0)
    return pl.pallas_call(
        _mlp_body,
        grid=(3 * _NB,),
        in_specs=[pl.BlockSpec((_MB, 3), blk),
                  pl.BlockSpec((_MB, F2), blk),
                  pl.BlockSpec((_MB, F1), blk),
                  pl.BlockSpec((3 + F2 + F1, H), cst),
                  pl.BlockSpec((1, H), cst),
                  pl.BlockSpec((1, H), cst),
                  pl.BlockSpec((1, H), cst),
                  pl.BlockSpec((H, H), cst),
                  pl.BlockSpec((1, H), cst),
                  pl.BlockSpec((1, H), cst),
                  pl.BlockSpec((1, H), cst)],
        out_specs=pl.BlockSpec((_MB, H), out_map),
        out_shape=jax.ShapeDtypeStruct((N_M, H), jnp.float32),
        scratch_shapes=[pltpu.VMEM((N_M, H), jnp.float32),
                        pltpu.VMEM((N_M, H), jnp.float32),
                        pltpu.VMEM((2, H), jnp.float32),
                        pltpu.VMEM((2, H), jnp.float32)],
        compiler_params=pltpu.CompilerParams(
            dimension_semantics=("arbitrary",)),
    )(coords, feats, interp, w0, b0, g0, be0, w1, b1, g1, be1)


def kernel(prop_coords, prop_feats, orig_coords, orig_feats,
           W0, b0, g0, be0, W1, b1, g1, be1):
    prop_t = prop_coords.T
    idx, w = _knn(orig_coords, prop_t)
    interp = _sc_interp(prop_feats, idx.reshape(-1), w.reshape(-1))
    return _mlp(orig_coords, orig_feats, interp, W0, b0.reshape(1, H),
                g0.reshape(1, H), be0.reshape(1, H), W1, b1.reshape(1, H),
                g1.reshape(1, H), be1.reshape(1, H))


# speedup vs baseline: 1.0429x; 1.0429x over previous
import functools

import jax
import jax.numpy as jnp
from jax import lax
from jax.experimental import pallas as pl
from jax.experimental.pallas import tpu as pltpu
from jax.experimental.pallas import tpu_sc as plsc

N_L = 4096
N_M = 16384
F1 = 64
F2 = 64
H = 128
_EPS_SQ = 1e-12
_BN_EPS = 1e-5

_QB = 256
_NQ = N_M // _QB
_MB = 512
_NB = N_M // _MB

_NW = 32
_QPW = N_M // _NW
_QCH = 256
_NCH = _QPW // _QCH
_RCH = 3 * _QCH


def _knn_body(q_ref, pt_ref, idx_ref, w_ref):
    q = q_ref[...]
    pt = pt_ref[...]
    d0 = q[:, 0:1] - pt[0:1, :]
    d1 = q[:, 1:2] - pt[1:2, :]
    d2 = q[:, 2:3] - pt[2:3, :]
    s = d0 * d0 + d1 * d1 + d2 * d2
    colsf = lax.broadcasted_iota(jnp.int32, s.shape, 1).astype(jnp.float32)
    big = jnp.float32(jnp.inf)
    m1 = jnp.min(s, axis=1, keepdims=True)
    m2 = jnp.min(jnp.where(s > m1, s, big), axis=1, keepdims=True)
    m3 = jnp.min(jnp.where(s > m2, s, big), axis=1, keepdims=True)
    ms = [m1, m2, m3]
    idxf = [jnp.min(jnp.where(s == m, colsf, big), axis=1, keepdims=True)
            for m in ms]
    w = [1.0 / jnp.maximum(m, jnp.float32(_EPS_SQ)) for m in ms]
    inv = 1.0 / (w[0] + w[1] + w[2])
    w_ref[...] = jnp.concatenate([wk * inv for wk in w], axis=1)
    idx_ref[...] = jnp.concatenate(
        [f.astype(jnp.int32) for f in idxf], axis=1)


def _knn(orig_coords, prop_t):
    return pl.pallas_call(
        _knn_body,
        grid=(_NQ,),
        in_specs=[pl.BlockSpec((_QB, 3), lambda i: (i, 0)),
                  pl.BlockSpec((3, N_L), lambda i: (0, 0))],
        out_specs=[pl.BlockSpec((_QB, 3), lambda i: (i, 0)),
                   pl.BlockSpec((_QB, 3), lambda i: (i, 0))],
        out_shape=[jax.ShapeDtypeStruct((N_M, 3), jnp.int32),
                   jax.ShapeDtypeStruct((N_M, 3), jnp.float32)],
    )(orig_coords, prop_t)


def _sc_interp(prop_feats, idx_flat, w_flat):
    mesh = plsc.VectorSubcoreMesh(core_axis_name="c", subcore_axis_name="s")

    @functools.partial(
        pl.kernel,
        mesh=mesh,
        out_type=jax.ShapeDtypeStruct((N_M, F1), jnp.float32),
        scratch_types=[
            pltpu.VMEM((_RCH,), jnp.int32),
            pltpu.VMEM((_RCH + 16,), jnp.float32),
            pltpu.VMEM((_RCH, F1), jnp.float32),
            pltpu.VMEM((_QCH, F1), jnp.float32),
            pltpu.SemaphoreType.DMA,
        ],
        compiler_params=pltpu.CompilerParams(use_tc_tiling_on_sc=False),
    )
    def k(feats_hbm, idx_hbm, wgt_hbm, out_hbm, idx_v, w_v, rows_v, out_v, sem):
        wid = lax.axis_index("s") * 2 + lax.axis_index("c")
        for ch in range(_NCH):
            qbase = wid * _QPW + ch * _QCH
            rbase = 3 * qbase
            pltpu.sync_copy(idx_hbm.at[pl.ds(rbase, _RCH)], idx_v)
            pltpu.sync_copy(wgt_hbm.at[pl.ds(rbase, _RCH)],
                            w_v.at[pl.ds(0, _RCH)])
            pltpu.async_copy(feats_hbm.at[idx_v], rows_v, sem).wait()

            def body(qq, carry):
                r = 3 * qq
                wv = w_v[pl.ds(r, 16)]
                w0 = wv[0]
                w1 = wv[1]
                w2 = wv[2]
                for cch in range(F1 // 16):
                    slc = pl.ds(cch * 16, 16)
                    out_v[qq, slc] = (rows_v[r, slc] * w0
                                      + rows_v[r + 1, slc] * w1
                                      + rows_v[r + 2, slc] * w2)
                return carry

            lax.fori_loop(0, _QCH, body, 0)
            pltpu.sync_copy(out_v, out_hbm.at[pl.ds(qbase, _QCH)])

    return k(prop_feats, idx_flat, w_flat)


def _bn(h, st, g, be):
    mu = st[0:1, :] * (1.0 / N_M)
    var = st[1:2, :] * (1.0 / N_M) - mu * mu
    v = var + jnp.float32(_BN_EPS)
    inv = lax.rsqrt(v)
    inv = inv * (1.5 - 0.5 * v * inv * inv)
    inv = inv * (1.5 - 0.5 * v * inv * inv)
    return jnp.maximum((h - mu) * inv * g + be, 0.0)


def _stats(h):
    return jnp.concatenate([jnp.sum(h, axis=0, keepdims=True),
                            jnp.sum(h * h, axis=0, keepdims=True)], axis=0)


def _mlp_body(c_ref, f_ref, it_ref, w0_ref, b0_ref, g0_ref, be0_ref,
              w1_ref, b1_ref, g1_ref, be1_ref, o_ref,
              h0_s, h1_s, acc0, acc1):
    step = pl.program_id(0)
    i = step % _NB
    rows = pl.ds(i * _MB, _MB)

    @pl.when(step < _NB)
    def _():
        x = jnp.concatenate([c_ref[...], f_ref[...], it_ref[...]], axis=1)
        h = jnp.dot(x, w0_ref[...],
                    preferred_element_type=jnp.float32) + b0_ref[...]
        h0_s[rows, :] = h

        @pl.when(step == 0)
        def _():
            acc0[...] = jnp.zeros_like(acc0)

        acc0[...] += _stats(h)

    @pl.when(jnp.logical_and(step >= _NB, step < 2 * _NB))
    def _():
        z = _bn(h0_s[rows, :], acc0[...], g0_ref[...], be0_ref[...])
        h1 = jnp.dot(z, w1_ref[...],
                     preferred_element_type=jnp.float32) + b1_ref[...]
        h1_s[rows, :] = h1

        @pl.when(step == _NB)
        def _():
            acc1[...] = jnp.zeros_like(acc1)

        acc1[...] += _stats(h1)

    @pl.when(step >= 2 * _NB)
    def _():
        o_ref[...] = _bn(h1_s[rows, :], acc1[...], g1_ref[...], be1_ref[...])


def _mlp(coords, feats, interp, w0, b0, g0, be0, w1, b1, g1, be1):
    blk = lambda g: (g % _NB, 0)
    cst = lambda g: (0, 0)
    out_map = lambda g: (jnp.where(g < 2 * _NB, 0, g % _NB), 0)
    return pl.pallas_call(
        _mlp_body,
        grid=(3 * _NB,),
        in_specs=[pl.BlockSpec((_MB, 3), blk),
                  pl.BlockSpec((_MB, F2), blk),
                  pl.BlockSpec((_MB, F1), blk),
                  pl.BlockSpec((3 + F2 + F1, H), cst),
                  pl.BlockSpec((1, H), cst),
                  pl.BlockSpec((1, H), cst),
                  pl.BlockSpec((1, H), cst),
                  pl.BlockSpec((H, H), cst),
                  pl.BlockSpec((1, H), cst),
                  pl.BlockSpec((1, H), cst),
                  pl.BlockSpec((1, H), cst)],
        out_specs=pl.BlockSpec((_MB, H), out_map),
        out_shape=jax.ShapeDtypeStruct((N_M, H), jnp.float32),
        scratch_shapes=[pltpu.VMEM((N_M, H), jnp.float32),
                        pltpu.VMEM((N_M, H), jnp.float32),
                        pltpu.VMEM((2, H), jnp.float32),
                        pltpu.VMEM((2, H), jnp.float32)],
        compiler_params=pltpu.CompilerParams(
            dimension_semantics=("arbitrary",)),
    )(coords, feats, interp, w0, b0, g0, be0, w1, b1, g1, be1)


def kernel(prop_coords, prop_feats, orig_coords, orig_feats,
           W0, b0, g0, be0, W1, b1, g1, be1):
    prop_t = prop_coords.T
    idx, w = _knn(orig_coords, prop_t)
    interp = _sc_interp(prop_feats, idx.reshape(-1), w.reshape(-1))
    return _mlp(orig_coords, orig_feats, interp, W0, b0.reshape(1, H),
                g0.reshape(1, H), be0.reshape(1, H), W1, b1.reshape(1, H),
                g1.reshape(1, H), be1.reshape(1, H))


# speedup vs baseline: 1.1551x; 1.1076x over previous
import functools

import jax
import jax.numpy as jnp
from jax import lax
from jax.experimental import pallas as pl
from jax.experimental.pallas import tpu as pltpu
from jax.experimental.pallas import tpu_sc as plsc

N_L = 4096
N_M = 16384
F1 = 64
F2 = 64
H = 128
_EPS_SQ = 1e-12
_BN_EPS = 1e-5

_QB = 256
_NQ = N_M // _QB
_MB = 2048
_NB = N_M // _MB

_NW = 32
_QPW = N_M // _NW
_QCH = 256
_NCH = _QPW // _QCH
_RCH = 3 * _QCH


def _knn_body(q_ref, pt_ref, idx_ref, w_ref):
    q = q_ref[...]
    pt = pt_ref[...]
    d0 = q[:, 0:1] - pt[0:1, :]
    d1 = q[:, 1:2] - pt[1:2, :]
    d2 = q[:, 2:3] - pt[2:3, :]
    s = d0 * d0 + d1 * d1 + d2 * d2
    colsf = lax.broadcasted_iota(jnp.int32, s.shape, 1).astype(jnp.float32)
    big = jnp.float32(jnp.inf)
    m1 = jnp.min(s, axis=1, keepdims=True)
    p1 = s > m1
    t1 = jnp.where(p1, s, big)
    i1 = jnp.min(jnp.where(p1, big, colsf), axis=1, keepdims=True)
    m2 = jnp.min(t1, axis=1, keepdims=True)
    p2 = t1 > m2
    t2 = jnp.where(p2, t1, big)
    i2 = jnp.min(jnp.where(p2, big, colsf), axis=1, keepdims=True)
    m3 = jnp.min(t2, axis=1, keepdims=True)
    i3 = jnp.min(jnp.where(t2 > m3, big, colsf), axis=1, keepdims=True)
    ms = [m1, m2, m3]
    idxf = [i1, i2, i3]
    w = [1.0 / jnp.maximum(m, jnp.float32(_EPS_SQ)) for m in ms]
    inv = 1.0 / (w[0] + w[1] + w[2])
    w_ref[...] = jnp.concatenate([wk * inv for wk in w], axis=1)
    idx_ref[...] = jnp.concatenate(
        [f.astype(jnp.int32) for f in idxf], axis=1)


def _knn(orig_coords, prop_t):
    return pl.pallas_call(
        _knn_body,
        grid=(_NQ,),
        in_specs=[pl.BlockSpec((_QB, 3), lambda i: (i, 0)),
                  pl.BlockSpec((3, N_L), lambda i: (0, 0))],
        out_specs=[pl.BlockSpec((_QB, 3), lambda i: (i, 0)),
                   pl.BlockSpec((_QB, 3), lambda i: (i, 0))],
        out_shape=[jax.ShapeDtypeStruct((N_M, 3), jnp.int32),
                   jax.ShapeDtypeStruct((N_M, 3), jnp.float32)],
    )(orig_coords, prop_t)


def _sc_interp(prop_feats, idx_flat, w_flat):
    mesh = plsc.VectorSubcoreMesh(core_axis_name="c", subcore_axis_name="s")

    @functools.partial(
        pl.kernel,
        mesh=mesh,
        out_type=jax.ShapeDtypeStruct((N_M, F1), jnp.float32),
        scratch_types=[
            pltpu.VMEM((_RCH,), jnp.int32),
            pltpu.VMEM((_RCH + 16,), jnp.float32),
            pltpu.VMEM((_RCH, F1), jnp.float32),
            pltpu.VMEM((_QCH, F1), jnp.float32),
            pltpu.SemaphoreType.DMA,
        ],
        compiler_params=pltpu.CompilerParams(use_tc_tiling_on_sc=False),
    )
    def k(feats_hbm, idx_hbm, wgt_hbm, out_hbm, idx_v, w_v, rows_v, out_v, sem):
        wid = lax.axis_index("s") * 2 + lax.axis_index("c")
        for ch in range(_NCH):
            qbase = wid * _QPW + ch * _QCH
            rbase = 3 * qbase
            pltpu.sync_copy(idx_hbm.at[pl.ds(rbase, _RCH)], idx_v)
            pltpu.sync_copy(wgt_hbm.at[pl.ds(rbase, _RCH)],
                            w_v.at[pl.ds(0, _RCH)])
            pltpu.async_copy(feats_hbm.at[idx_v], rows_v, sem).wait()

            def body(qq, carry):
                r = 3 * qq
                wv = w_v[pl.ds(r, 16)]
                w0 = wv[0]
                w1 = wv[1]
                w2 = wv[2]
                for cch in range(F1 // 16):
                    slc = pl.ds(cch * 16, 16)
                    out_v[qq, slc] = (rows_v[r, slc] * w0
                                      + rows_v[r + 1, slc] * w1
                                      + rows_v[r + 2, slc] * w2)
                return carry

            lax.fori_loop(0, _QCH, body, 0)
            pltpu.sync_copy(out_v, out_hbm.at[pl.ds(qbase, _QCH)])

    return k(prop_feats, idx_flat, w_flat)


def _bn(h, st, g, be):
    mu = st[0:1, :] * (1.0 / N_M)
    var = st[1:2, :] * (1.0 / N_M) - mu * mu
    v = var + jnp.float32(_BN_EPS)
    inv = lax.rsqrt(v)
    inv = inv * (1.5 - 0.5 * v * inv * inv)
    inv = inv * (1.5 - 0.5 * v * inv * inv)
    return jnp.maximum((h - mu) * inv * g + be, 0.0)


def _stats(h):
    return jnp.concatenate([jnp.sum(h, axis=0, keepdims=True),
                            jnp.sum(h * h, axis=0, keepdims=True)], axis=0)


def _mlp_body(c_ref, f_ref, it_ref, w0_ref, b0_ref, g0_ref, be0_ref,
              w1_ref, b1_ref, g1_ref, be1_ref, o_ref,
              h0_s, h1_s, acc0, acc1):
    step = pl.program_id(0)
    i = step % _NB
    rows = pl.ds(i * _MB, _MB)

    @pl.when(step < _NB)
    def _():
        x = jnp.concatenate([c_ref[...], f_ref[...], it_ref[...]], axis=1)
        h = jnp.dot(x, w0_ref[...],
                    preferred_element_type=jnp.float32) + b0_ref[...]
        h0_s[rows, :] = h

        @pl.when(step == 0)
        def _():
            acc0[...] = jnp.zeros_like(acc0)

        acc0[...] += _stats(h)

    @pl.when(jnp.logical_and(step >= _NB, step < 2 * _NB))
    def _():
        z = _bn(h0_s[rows, :], acc0[...], g0_ref[...], be0_ref[...])
        h1 = jnp.dot(z, w1_ref[...],
                     preferred_element_type=jnp.float32) + b1_ref[...]
        h1_s[rows, :] = h1

        @pl.when(step == _NB)
        def _():
            acc1[...] = jnp.zeros_like(acc1)

        acc1[...] += _stats(h1)

    @pl.when(step >= 2 * _NB)
    def _():
        o_ref[...] = _bn(h1_s[rows, :], acc1[...], g1_ref[...], be1_ref[...])


def _mlp(coords, feats, interp, w0, b0, g0, be0, w1, b1, g1, be1):
    blk = lambda g: (g % _NB, 0)
    cst = lambda g: (0, 0)
    out_map = lambda g: (jnp.where(g < 2 * _NB, 0, g % _NB), 0)
    return pl.pallas_call(
        _mlp_body,
        grid=(3 * _NB,),
        in_specs=[pl.BlockSpec((_MB, 3), blk),
                  pl.BlockSpec((_MB, F2), blk),
                  pl.BlockSpec((_MB, F1), blk),
                  pl.BlockSpec((3 + F2 + F1, H), cst),
                  pl.BlockSpec((1, H), cst),
                  pl.BlockSpec((1, H), cst),
                  pl.BlockSpec((1, H), cst),
                  pl.BlockSpec((H, H), cst),
                  pl.BlockSpec((1, H), cst),
                  pl.BlockSpec((1, H), cst),
                  pl.BlockSpec((1, H), cst)],
        out_specs=pl.BlockSpec((_MB, H), out_map),
        out_shape=jax.ShapeDtypeStruct((N_M, H), jnp.float32),
        scratch_shapes=[pltpu.VMEM((N_M, H), jnp.float32),
                        pltpu.VMEM((N_M, H), jnp.float32),
                        pltpu.VMEM((2, H), jnp.float32),
                        pltpu.VMEM((2, H), jnp.float32)],
        compiler_params=pltpu.CompilerParams(
            dimension_semantics=("arbitrary",)),
    )(coords, feats, interp, w0, b0, g0, be0, w1, b1, g1, be1)


def kernel(prop_coords, prop_feats, orig_coords, orig_feats,
           W0, b0, g0, be0, W1, b1, g1, be1):
    prop_t = prop_coords.T
    idx, w = _knn(orig_coords, prop_t)
    interp = _sc_interp(prop_feats, idx.reshape(-1), w.reshape(-1))
    return _mlp(orig_coords, orig_feats, interp, W0, b0.reshape(1, H),
                g0.reshape(1, H), be0.reshape(1, H), W1, b1.reshape(1, H),
                g1.reshape(1, H), be1.reshape(1, H))


# speedup vs baseline: 1.1734x; 1.0158x over previous
import functools

import jax
import jax.numpy as jnp
from jax import lax
from jax.experimental import pallas as pl
from jax.experimental.pallas import tpu as pltpu
from jax.experimental.pallas import tpu_sc as plsc

N_L = 4096
N_M = 16384
F1 = 64
F2 = 64
H = 128
_EPS_SQ = 1e-12
_BN_EPS = 1e-5

_QB = 256
_NQ = N_M // _QB
_MB = 4096
_NB = N_M // _MB

_NW = 32
_QPW = N_M // _NW
_QCH = 256
_NCH = _QPW // _QCH
_RCH = 3 * _QCH


def _knn_body(q_ref, pt_ref, idx_ref, w_ref):
    q = q_ref[...]
    pt = pt_ref[...]
    d0 = q[:, 0:1] - pt[0:1, :]
    d1 = q[:, 1:2] - pt[1:2, :]
    d2 = q[:, 2:3] - pt[2:3, :]
    s = d0 * d0 + d1 * d1 + d2 * d2
    colsf = lax.broadcasted_iota(jnp.int32, s.shape, 1).astype(jnp.float32)
    big = jnp.float32(jnp.inf)
    m1 = jnp.min(s, axis=1, keepdims=True)
    p1 = s > m1
    t1 = jnp.where(p1, s, big)
    i1 = jnp.min(jnp.where(p1, big, colsf), axis=1, keepdims=True)
    m2 = jnp.min(t1, axis=1, keepdims=True)
    p2 = t1 > m2
    t2 = jnp.where(p2, t1, big)
    i2 = jnp.min(jnp.where(p2, big, colsf), axis=1, keepdims=True)
    m3 = jnp.min(t2, axis=1, keepdims=True)
    i3 = jnp.min(jnp.where(t2 > m3, big, colsf), axis=1, keepdims=True)
    ms = [m1, m2, m3]
    idxf = [i1, i2, i3]
    w = [1.0 / jnp.maximum(m, jnp.float32(_EPS_SQ)) for m in ms]
    inv = 1.0 / (w[0] + w[1] + w[2])
    w_ref[...] = jnp.concatenate([wk * inv for wk in w], axis=1)
    idx_ref[...] = jnp.concatenate(
        [f.astype(jnp.int32) for f in idxf], axis=1)


def _knn(orig_coords, prop_t):
    return pl.pallas_call(
        _knn_body,
        grid=(_NQ,),
        in_specs=[pl.BlockSpec((_QB, 3), lambda i: (i, 0)),
                  pl.BlockSpec((3, N_L), lambda i: (0, 0))],
        out_specs=[pl.BlockSpec((_QB, 3), lambda i: (i, 0)),
                   pl.BlockSpec((_QB, 3), lambda i: (i, 0))],
        out_shape=[jax.ShapeDtypeStruct((N_M, 3), jnp.int32),
                   jax.ShapeDtypeStruct((N_M, 3), jnp.float32)],
    )(orig_coords, prop_t)


def _sc_interp(prop_feats, idx_flat, w_flat):
    mesh = plsc.VectorSubcoreMesh(core_axis_name="c", subcore_axis_name="s")

    @functools.partial(
        pl.kernel,
        mesh=mesh,
        out_type=jax.ShapeDtypeStruct((N_M, F1), jnp.float32),
        scratch_types=[
            pltpu.VMEM((_RCH,), jnp.int32),
            pltpu.VMEM((_RCH + 16,), jnp.float32),
            pltpu.VMEM((_RCH, F1), jnp.float32),
            pltpu.VMEM((_QCH, F1), jnp.float32),
            pltpu.SemaphoreType.DMA,
        ],
        compiler_params=pltpu.CompilerParams(use_tc_tiling_on_sc=False),
    )
    def k(feats_hbm, idx_hbm, wgt_hbm, out_hbm, idx_v, w_v, rows_v, out_v, sem):
        wid = lax.axis_index("s") * 2 + lax.axis_index("c")
        for ch in range(_NCH):
            qbase = wid * _QPW + ch * _QCH
            rbase = 3 * qbase
            pltpu.sync_copy(idx_hbm.at[pl.ds(rbase, _RCH)], idx_v)
            pltpu.sync_copy(wgt_hbm.at[pl.ds(rbase, _RCH)],
                            w_v.at[pl.ds(0, _RCH)])
            pltpu.async_copy(feats_hbm.at[idx_v], rows_v, sem).wait()

            def body(qq, carry):
                r = 3 * qq
                wv = w_v[pl.ds(r, 16)]
                w0 = wv[0]
                w1 = wv[1]
                w2 = wv[2]
                for cch in range(F1 // 16):
                    slc = pl.ds(cch * 16, 16)
                    out_v[qq, slc] = (rows_v[r, slc] * w0
                                      + rows_v[r + 1, slc] * w1
                                      + rows_v[r + 2, slc] * w2)
                return carry

            lax.fori_loop(0, _QCH, body, 0)
            pltpu.sync_copy(out_v, out_hbm.at[pl.ds(qbase, _QCH)])

    return k(prop_feats, idx_flat, w_flat)


def _bn(h, st, g, be):
    mu = st[0:1, :] * (1.0 / N_M)
    var = st[1:2, :] * (1.0 / N_M) - mu * mu
    v = var + jnp.float32(_BN_EPS)
    inv = lax.rsqrt(v)
    inv = inv * (1.5 - 0.5 * v * inv * inv)
    inv = inv * (1.5 - 0.5 * v * inv * inv)
    return jnp.maximum((h - mu) * inv * g + be, 0.0)


def _stats(h):
    return jnp.concatenate([jnp.sum(h, axis=0, keepdims=True),
                            jnp.sum(h * h, axis=0, keepdims=True)], axis=0)


def _mlp_body(c_ref, f_ref, it_ref, w0_ref, b0_ref, g0_ref, be0_ref,
              w1_ref, b1_ref, g1_ref, be1_ref, o_ref,
              h0_s, h1_s, acc0, acc1):
    step = pl.program_id(0)
    i = step % _NB
    rows = pl.ds(i * _MB, _MB)

    @pl.when(step < _NB)
    def _():
        x = jnp.concatenate([c_ref[...], f_ref[...], it_ref[...]], axis=1)
        h = jnp.dot(x, w0_ref[...],
                    preferred_element_type=jnp.float32) + b0_ref[...]
        h0_s[rows, :] = h

        @pl.when(step == 0)
        def _():
            acc0[...] = jnp.zeros_like(acc0)

        acc0[...] += _stats(h)

    @pl.when(jnp.logical_and(step >= _NB, step < 2 * _NB))
    def _():
        z = _bn(h0_s[rows, :], acc0[...], g0_ref[...], be0_ref[...])
        h1 = jnp.dot(z, w1_ref[...],
                     preferred_element_type=jnp.float32) + b1_ref[...]
        h1_s[rows, :] = h1

        @pl.when(step == _NB)
        def _():
            acc1[...] = jnp.zeros_like(acc1)

        acc1[...] += _stats(h1)

    @pl.when(step >= 2 * _NB)
    def _():
        o_ref[...] = _bn(h1_s[rows, :], acc1[...], g1_ref[...], be1_ref[...])


def _mlp(coords, feats, interp, w0, b0, g0, be0, w1, b1, g1, be1):
    blk = lambda g: (g % _NB, 0)
    cst = lambda g: (0, 0)
    out_map = lambda g: (jnp.where(g < 2 * _NB, 0, g % _NB), 0)
    return pl.pallas_call(
        _mlp_body,
        grid=(3 * _NB,),
        in_specs=[pl.BlockSpec((_MB, 3), blk),
                  pl.BlockSpec((_MB, F2), blk),
                  pl.BlockSpec((_MB, F1), blk),
                  pl.BlockSpec((3 + F2 + F1, H), cst),
                  pl.BlockSpec((1, H), cst),
                  pl.BlockSpec((1, H), cst),
                  pl.BlockSpec((1, H), cst),
                  pl.BlockSpec((H, H), cst),
                  pl.BlockSpec((1, H), cst),
                  pl.BlockSpec((1, H), cst),
                  pl.BlockSpec((1, H), cst)],
        out_specs=pl.BlockSpec((_MB, H), out_map),
        out_shape=jax.ShapeDtypeStruct((N_M, H), jnp.float32),
        scratch_shapes=[pltpu.VMEM((N_M, H), jnp.float32),
                        pltpu.VMEM((N_M, H), jnp.float32),
                        pltpu.VMEM((2, H), jnp.float32),
                        pltpu.VMEM((2, H), jnp.float32)],
        compiler_params=pltpu.CompilerParams(
            dimension_semantics=("arbitrary",)),
    )(coords, feats, interp, w0, b0, g0, be0, w1, b1, g1, be1)


def kernel(prop_coords, prop_feats, orig_coords, orig_feats,
           W0, b0, g0, be0, W1, b1, g1, be1):
    prop_t = prop_coords.T
    idx, w = _knn(orig_coords, prop_t)
    interp = _sc_interp(prop_feats, idx.reshape(-1), w.reshape(-1))
    return _mlp(orig_coords, orig_feats, interp, W0, b0.reshape(1, H),
                g0.reshape(1, H), be0.reshape(1, H), W1, b1.reshape(1, H),
                g1.reshape(1, H), be1.reshape(1, H))


# speedup vs baseline: 1.1768x; 1.0029x over previous
import functools

import jax
import jax.numpy as jnp
from jax import lax
from jax.experimental import pallas as pl
from jax.experimental.pallas import tpu as pltpu
from jax.experimental.pallas import tpu_sc as plsc

N_L = 4096
N_M = 16384
F1 = 64
F2 = 64
H = 128
_EPS_SQ = 1e-12
_BN_EPS = 1e-5

_QB = 256
_NQ = N_M // _QB
_MB = 8192
_NB = N_M // _MB

_NW = 32
_QPW = N_M // _NW
_QCH = 256
_NCH = _QPW // _QCH
_RCH = 3 * _QCH


def _knn_body(q_ref, pt_ref, idx_ref, w_ref):
    q = q_ref[...]
    pt = pt_ref[...]
    d0 = q[:, 0:1] - pt[0:1, :]
    d1 = q[:, 1:2] - pt[1:2, :]
    d2 = q[:, 2:3] - pt[2:3, :]
    s = d0 * d0 + d1 * d1 + d2 * d2
    colsf = lax.broadcasted_iota(jnp.int32, s.shape, 1).astype(jnp.float32)
    big = jnp.float32(jnp.inf)
    m1 = jnp.min(s, axis=1, keepdims=True)
    p1 = s > m1
    t1 = jnp.where(p1, s, big)
    i1 = jnp.min(jnp.where(p1, big, colsf), axis=1, keepdims=True)
    m2 = jnp.min(t1, axis=1, keepdims=True)
    p2 = t1 > m2
    t2 = jnp.where(p2, t1, big)
    i2 = jnp.min(jnp.where(p2, big, colsf), axis=1, keepdims=True)
    m3 = jnp.min(t2, axis=1, keepdims=True)
    i3 = jnp.min(jnp.where(t2 > m3, big, colsf), axis=1, keepdims=True)
    ms = [m1, m2, m3]
    idxf = [i1, i2, i3]
    w = [1.0 / jnp.maximum(m, jnp.float32(_EPS_SQ)) for m in ms]
    inv = 1.0 / (w[0] + w[1] + w[2])
    w_ref[...] = jnp.concatenate([wk * inv for wk in w], axis=1)
    idx_ref[...] = jnp.concatenate(
        [f.astype(jnp.int32) for f in idxf], axis=1)


def _knn(orig_coords, prop_t):
    return pl.pallas_call(
        _knn_body,
        grid=(_NQ,),
        in_specs=[pl.BlockSpec((_QB, 3), lambda i: (i, 0)),
                  pl.BlockSpec((3, N_L), lambda i: (0, 0))],
        out_specs=[pl.BlockSpec((_QB, 3), lambda i: (i, 0)),
                   pl.BlockSpec((_QB, 3), lambda i: (i, 0))],
        out_shape=[jax.ShapeDtypeStruct((N_M, 3), jnp.int32),
                   jax.ShapeDtypeStruct((N_M, 3), jnp.float32)],
    )(orig_coords, prop_t)


def _sc_interp(prop_feats, idx_flat, w_flat):
    mesh = plsc.VectorSubcoreMesh(core_axis_name="c", subcore_axis_name="s")

    @functools.partial(
        pl.kernel,
        mesh=mesh,
        out_type=jax.ShapeDtypeStruct((N_M, F1), jnp.float32),
        scratch_types=[
            pltpu.VMEM((_RCH,), jnp.int32),
            pltpu.VMEM((_RCH + 16,), jnp.float32),
            pltpu.VMEM((_RCH, F1), jnp.float32),
            pltpu.VMEM((_QCH, F1), jnp.float32),
            pltpu.SemaphoreType.DMA,
        ],
        compiler_params=pltpu.CompilerParams(use_tc_tiling_on_sc=False),
    )
    def k(feats_hbm, idx_hbm, wgt_hbm, out_hbm, idx_v, w_v, rows_v, out_v, sem):
        wid = lax.axis_index("s") * 2 + lax.axis_index("c")
        for ch in range(_NCH):
            qbase = wid * _QPW + ch * _QCH
            rbase = 3 * qbase
            pltpu.sync_copy(idx_hbm.at[pl.ds(rbase, _RCH)], idx_v)
            pltpu.sync_copy(wgt_hbm.at[pl.ds(rbase, _RCH)],
                            w_v.at[pl.ds(0, _RCH)])
            pltpu.async_copy(feats_hbm.at[idx_v], rows_v, sem).wait()

            def body(qq, carry):
                r = 3 * qq
                wv = w_v[pl.ds(r, 16)]
                w0 = wv[0]
                w1 = wv[1]
                w2 = wv[2]
                for cch in range(F1 // 16):
                    slc = pl.ds(cch * 16, 16)
                    out_v[qq, slc] = (rows_v[r, slc] * w0
                                      + rows_v[r + 1, slc] * w1
                                      + rows_v[r + 2, slc] * w2)
                return carry

            lax.fori_loop(0, _QCH, body, 0)
            pltpu.sync_copy(out_v, out_hbm.at[pl.ds(qbase, _QCH)])

    return k(prop_feats, idx_flat, w_flat)


def _bn(h, st, g, be):
    mu = st[0:1, :] * (1.0 / N_M)
    var = st[1:2, :] * (1.0 / N_M) - mu * mu
    v = var + jnp.float32(_BN_EPS)
    inv = lax.rsqrt(v)
    inv = inv * (1.5 - 0.5 * v * inv * inv)
    inv = inv * (1.5 - 0.5 * v * inv * inv)
    return jnp.maximum((h - mu) * inv * g + be, 0.0)


def _stats(h):
    return jnp.concatenate([jnp.sum(h, axis=0, keepdims=True),
                            jnp.sum(h * h, axis=0, keepdims=True)], axis=0)


def _mlp_body(c_ref, f_ref, it_ref, w0_ref, b0_ref, g0_ref, be0_ref,
              w1_ref, b1_ref, g1_ref, be1_ref, o_ref,
              h0_s, h1_s, acc0, acc1):
    step = pl.program_id(0)
    i = step % _NB
    rows = pl.ds(i * _MB, _MB)

    @pl.when(step < _NB)
    def _():
        x = jnp.concatenate([c_ref[...], f_ref[...], it_ref[...]], axis=1)
        h = jnp.dot(x, w0_ref[...],
                    preferred_element_type=jnp.float32) + b0_ref[...]
        h0_s[rows, :] = h

        @pl.when(step == 0)
        def _():
            acc0[...] = jnp.zeros_like(acc0)

        acc0[...] += _stats(h)

    @pl.when(jnp.logical_and(step >= _NB, step < 2 * _NB))
    def _():
        z = _bn(h0_s[rows, :], acc0[...], g0_ref[...], be0_ref[...])
        h1 = jnp.dot(z, w1_ref[...],
                     preferred_element_type=jnp.float32) + b1_ref[...]
        h1_s[rows, :] = h1

        @pl.when(step == _NB)
        def _():
            acc1[...] = jnp.zeros_like(acc1)

        acc1[...] += _stats(h1)

    @pl.when(step >= 2 * _NB)
    def _():
        o_ref[...] = _bn(h1_s[rows, :], acc1[...], g1_ref[...], be1_ref[...])


def _mlp(coords, feats, interp, w0, b0, g0, be0, w1, b1, g1, be1):
    blk = lambda g: (g % _NB, 0)
    cst = lambda g: (0, 0)
    out_map = lambda g: (jnp.where(g < 2 * _NB, 0, g % _NB), 0)
    return pl.pallas_call(
        _mlp_body,
        grid=(3 * _NB,),
        in_specs=[pl.BlockSpec((_MB, 3), blk),
                  pl.BlockSpec((_MB, F2), blk),
                  pl.BlockSpec((_MB, F1), blk),
                  pl.BlockSpec((3 + F2 + F1, H), cst),
                  pl.BlockSpec((1, H), cst),
                  pl.BlockSpec((1, H), cst),
                  pl.BlockSpec((1, H), cst),
                  pl.BlockSpec((H, H), cst),
                  pl.BlockSpec((1, H), cst),
                  pl.BlockSpec((1, H), cst),
                  pl.BlockSpec((1, H), cst)],
        out_specs=pl.BlockSpec((_MB, H), out_map),
        out_shape=jax.ShapeDtypeStruct((N_M, H), jnp.float32),
        scratch_shapes=[pltpu.VMEM((N_M, H), jnp.float32),
                        pltpu.VMEM((N_M, H), jnp.float32),
                        pltpu.VMEM((2, H), jnp.float32),
                        pltpu.VMEM((2, H), jnp.float32)],
        compiler_params=pltpu.CompilerParams(
            dimension_semantics=("arbitrary",)),
    )(coords, feats, interp, w0, b0, g0, be0, w1, b1, g1, be1)


def kernel(prop_coords, prop_feats, orig_coords, orig_feats,
           W0, b0, g0, be0, W1, b1, g1, be1):
    prop_t = prop_coords.T
    idx, w = _knn(orig_coords, prop_t)
    interp = _sc_interp(prop_feats, idx.reshape(-1), w.reshape(-1))
    return _mlp(orig_coords, orig_feats, interp, W0, b0.reshape(1, H),
                g0.reshape(1, H), be0.reshape(1, H), W1, b1.reshape(1, H),
                g1.reshape(1, H), be1.reshape(1, H))
